# baseline (speedup 1.0000x reference)
import jax
import jax.numpy as jnp
from jax.experimental import pallas as pl


def _body(w_ref, o_ref):
    o_ref[...] = w_ref[0:1, :]


def kernel(W):
    return pl.pallas_call(
        _body,
        out_shape=jax.ShapeDtypeStruct((1, W.shape[1]), W.dtype),
    )(W)
